# Initial kernel scaffold; baseline (speedup 1.0000x reference)
import jax
import jax.numpy as jnp
from jax.experimental import pallas as pl


def kernel(x, tau):
    raise NotImplementedError("write your pallas kernel here")


# speedup vs baseline: 5.4415x; 5.4415x over previous
import functools

import jax
import jax.numpy as jnp
import numpy as np
from jax import lax
from jax.experimental import pallas as pl
from jax.experimental.pallas import tpu as pltpu
from jax.experimental.pallas import tpu_sc as plsc

K = 128
HW = 1024
L = 16
NV = HW // L
NROWS = 32 * 384
NC = 2
NS = 16
NW = NC * NS
ROWS_PER_W = NROWS // NW
CH = 16
NCHUNK = ROWS_PER_W // CH
IMIN = np.int32(-2**31)

_mesh = plsc.VectorSubcoreMesh(core_axis_name="c", subcore_axis_name="s")


def _lane_perms():
    lanes = lax.iota(jnp.int32, L)
    return [lanes ^ jnp.int32(1 << p) for p in range(4)]


_GATHER_DNUMS = lax.GatherDimensionNumbers(
    offset_dims=(), collapsed_slice_dims=(0,), start_index_map=(0,)
)


def _permute(v, p):
    return lax.gather(
        v,
        p[:, None],
        _GATHER_DNUMS,
        slice_sizes=(1,),
        mode=lax.GatherScatterMode.PROMISE_IN_BOUNDS,
    )


def _allsum(v, perms):
    for p in perms:
        v = v + _permute(v, p)
    return v


@functools.partial(
    pl.kernel,
    out_type=jax.ShapeDtypeStruct((NROWS, HW), jnp.float32),
    mesh=_mesh,
    scratch_types=[
        pltpu.VMEM((CH, HW), jnp.float32),
        pltpu.VMEM((CH, HW), jnp.float32),
        pltpu.VMEM((HW,), jnp.int32),
    ],
)
def _topk_mask_mean(x_hbm, out_hbm, in_v, out_v, keys_v):
    wid = lax.axis_index("s") * NC + lax.axis_index("c")
    base_row = wid * ROWS_PER_W
    imin_v = jnp.full((L,), IMIN, jnp.int32)
    k_f = jnp.full((L,), np.float32(K), jnp.float32)
    one_f = jnp.ones((L,), jnp.float32)
    zero_f = jnp.zeros((L,), jnp.float32)
    perms = _lane_perms()

    def row_body(r, _):
        for j in range(NV):
            xv = in_v[r, pl.ds(j * L, L)]
            bv = lax.bitcast_convert_type(xv, jnp.int32)
            keys_v[pl.ds(j * L, L)] = jnp.where(bv >= 0, bv, imin_v - bv)

        def bit_body(_i, carry):
            prefix_v, bit_v = carry
            cand_v = prefix_v | bit_v
            ckey_v = cand_v ^ imin_v
            acc0 = jnp.zeros((L,), jnp.float32)
            acc1 = jnp.zeros((L,), jnp.float32)
            for j in range(0, NV, 2):
                acc0 = acc0 + jnp.where(keys_v[pl.ds(j * L, L)] >= ckey_v, one_f, zero_f)
                acc1 = acc1 + jnp.where(keys_v[pl.ds((j + 1) * L, L)] >= ckey_v, one_f, zero_f)
            cnt_v = _allsum(acc0 + acc1, perms)
            prefix_v = jnp.where(cnt_v >= k_f, cand_v, prefix_v)
            return prefix_v, lax.shift_right_logical(bit_v, 1)

        prefix_v, _bv = lax.fori_loop(
            0, 32, bit_body, (jnp.zeros((L,), jnp.int32), imin_v)
        )

        tk_v = prefix_v ^ imin_v
        tb_v = jnp.where(tk_v >= 0, tk_v, imin_v - tk_v)
        tf_v = lax.bitcast_convert_type(tb_v, jnp.float32)

        accs = jnp.zeros((L,), jnp.float32)
        accc = jnp.zeros((L,), jnp.float32)
        for j in range(NV):
            xv = in_v[r, pl.ds(j * L, L)]
            m = xv > tf_v
            accs = accs + jnp.where(m, xv, zero_f)
            accc = accc + jnp.where(m, one_f, zero_f)
        sum_gt = _allsum(accs, perms)
        cnt_gt = _allsum(accc, perms)
        mean_v = (sum_gt + (k_f - cnt_gt) * tf_v) * jnp.float32(1.0 / K)

        for j in range(NV):
            xv = in_v[r, pl.ds(j * L, L)]
            out_v[r, pl.ds(j * L, L)] = jnp.where(xv >= tf_v, mean_v, zero_f)
        return _

    def chunk_body(ci, _):
        row0 = base_row + ci * CH
        pltpu.sync_copy(x_hbm.at[pl.ds(row0, CH)], in_v)
        lax.fori_loop(0, CH, row_body, 0)
        pltpu.sync_copy(out_v, out_hbm.at[pl.ds(row0, CH)])
        return _

    lax.fori_loop(0, NCHUNK, chunk_body, 0)


def kernel(x, tau):
    b, c, h, w = x.shape
    out = _topk_mask_mean(x.reshape(b * c, h * w))
    return out.reshape(b, c, h, w)


# speedup vs baseline: 12.6553x; 2.3257x over previous
import functools

import jax
import jax.numpy as jnp
import numpy as np
from jax import lax
from jax.experimental import pallas as pl
from jax.experimental.pallas import tpu as pltpu
from jax.experimental.pallas import tpu_sc as plsc

K = 128
HW = 1024
L = 16
NV = HW // L
NROWS = 32 * 384
NC = 2
NS = 16
NW = NC * NS
CH = 16
IMIN = np.int32(-2**31)

SC_ROWS = 3584
TC_BLK = 256

_mesh = plsc.VectorSubcoreMesh(core_axis_name="c", subcore_axis_name="s")

_GATHER_DNUMS = lax.GatherDimensionNumbers(
    offset_dims=(), collapsed_slice_dims=(0,), start_index_map=(0,)
)


def _permute(v, p):
    return lax.gather(
        v,
        p[:, None],
        _GATHER_DNUMS,
        slice_sizes=(1,),
        mode=lax.GatherScatterMode.PROMISE_IN_BOUNDS,
    )


def _allsum(v, perms):
    for p in perms:
        v = v + _permute(v, p)
    return v


def _sc_body(x_hbm, out_hbm, in_v, out_v, keys_v):
    rows_per_w = SC_ROWS // NW
    nchunk = rows_per_w // CH
    wid = lax.axis_index("s") * NC + lax.axis_index("c")
    base_row = wid * rows_per_w
    imin_v = jnp.full((L,), IMIN, jnp.int32)
    k_f = jnp.full((L,), np.float32(K), jnp.float32)
    one_f = jnp.ones((L,), jnp.float32)
    zero_f = jnp.zeros((L,), jnp.float32)
    lanes = lax.iota(jnp.int32, L)
    perms = [lanes ^ jnp.int32(1 << p) for p in range(4)]

    def row_body(r, _):
        for j in range(NV):
            xv = in_v[r, pl.ds(j * L, L)]
            bv = lax.bitcast_convert_type(xv, jnp.int32)
            keys_v[pl.ds(j * L, L)] = jnp.where(bv >= 0, bv, imin_v - bv)

        def bit_body(_i, carry):
            prefix_v, bit_v = carry
            cand_v = prefix_v | bit_v
            ckey_v = cand_v ^ imin_v
            acc0 = jnp.zeros((L,), jnp.float32)
            acc1 = jnp.zeros((L,), jnp.float32)
            for j in range(0, NV, 2):
                acc0 = acc0 + jnp.where(keys_v[pl.ds(j * L, L)] >= ckey_v, one_f, zero_f)
                acc1 = acc1 + jnp.where(keys_v[pl.ds((j + 1) * L, L)] >= ckey_v, one_f, zero_f)
            cnt_v = _allsum(acc0 + acc1, perms)
            prefix_v = jnp.where(cnt_v >= k_f, cand_v, prefix_v)
            return prefix_v, lax.shift_right_logical(bit_v, 1)

        prefix_v, _bv = lax.fori_loop(
            0, 32, bit_body, (jnp.zeros((L,), jnp.int32), imin_v)
        )

        tk_v = prefix_v ^ imin_v
        tb_v = jnp.where(tk_v >= 0, tk_v, imin_v - tk_v)
        tf_v = lax.bitcast_convert_type(tb_v, jnp.float32)

        accs = jnp.zeros((L,), jnp.float32)
        accc = jnp.zeros((L,), jnp.float32)
        for j in range(NV):
            xv = in_v[r, pl.ds(j * L, L)]
            m = xv > tf_v
            accs = accs + jnp.where(m, xv, zero_f)
            accc = accc + jnp.where(m, one_f, zero_f)
        sum_gt = _allsum(accs, perms)
        cnt_gt = _allsum(accc, perms)
        mean_v = (sum_gt + (k_f - cnt_gt) * tf_v) * jnp.float32(1.0 / K)

        for j in range(NV):
            xv = in_v[r, pl.ds(j * L, L)]
            out_v[r, pl.ds(j * L, L)] = jnp.where(xv >= tf_v, mean_v, zero_f)
        return _

    def chunk_body(ci, _):
        row0 = base_row + ci * CH
        pltpu.sync_copy(x_hbm.at[pl.ds(row0, CH)], in_v)
        lax.fori_loop(0, CH, row_body, 0)
        pltpu.sync_copy(out_v, out_hbm.at[pl.ds(row0, CH)])
        return _

    lax.fori_loop(0, nchunk, chunk_body, 0)


_topk_sc = functools.partial(
    pl.kernel,
    out_type=jax.ShapeDtypeStruct((SC_ROWS, HW), jnp.float32),
    mesh=_mesh,
    scratch_types=[
        pltpu.VMEM((CH, HW), jnp.float32),
        pltpu.VMEM((CH, HW), jnp.float32),
        pltpu.VMEM((HW,), jnp.int32),
    ],
)(_sc_body)


def _tc_body(x_ref, o_ref):
    x = x_ref[...]
    bts = lax.bitcast_convert_type(x, jnp.int32)
    keys = jnp.where(bts >= 0, bts, IMIN - bts)
    r = x.shape[0]

    def bit_body(_i, carry):
        prefix, bit = carry
        cand = prefix | bit
        ck = cand ^ IMIN
        cnt = jnp.sum((keys >= ck).astype(jnp.int32), axis=1, keepdims=True)
        prefix = jnp.where(cnt >= K, cand, prefix)
        return prefix, lax.shift_right_logical(bit, 1)

    prefix, _bv = lax.fori_loop(
        0, 32, bit_body,
        (jnp.zeros((r, 1), jnp.int32), jnp.full((r, 1), IMIN, jnp.int32)),
    )
    tk = prefix ^ IMIN
    tb = jnp.where(tk >= 0, tk, IMIN - tk)
    t = lax.bitcast_convert_type(tb, jnp.float32)
    m_gt = x > t
    sum_gt = jnp.sum(jnp.where(m_gt, x, 0.0), axis=1, keepdims=True)
    cnt_gt = jnp.sum(m_gt.astype(jnp.int32), axis=1, keepdims=True)
    mean = (sum_gt + (np.float32(K) - cnt_gt.astype(jnp.float32)) * t) * np.float32(1.0 / K)
    o_ref[...] = jnp.where(x >= t, mean, 0.0)


def _topk_tc(xr):
    n = xr.shape[0]
    return pl.pallas_call(
        _tc_body,
        grid=(n // TC_BLK,),
        in_specs=[pl.BlockSpec((TC_BLK, HW), lambda i: (i, 0))],
        out_specs=pl.BlockSpec((TC_BLK, HW), lambda i: (i, 0)),
        out_shape=jax.ShapeDtypeStruct((n, HW), jnp.float32),
    )(xr)


def kernel(x, tau):
    b, c, h, w = x.shape
    xr = x.reshape(b * c, h * w)
    out_sc = _topk_sc(xr[:SC_ROWS])
    out_tc = _topk_tc(xr[SC_ROWS:])
    out = jnp.concatenate([out_sc, out_tc], axis=0)
    return out.reshape(b, c, h, w)
